# baseline (speedup 1.0000x reference)
import functools

import jax
import jax.numpy as jnp
from jax import lax
from jax.experimental import pallas as pl
from jax.experimental.pallas import tpu as pltpu
from jax.experimental.pallas import tpu_sc as plsc

N = 10000
E = 320000
D = 128
R = 8
NR = N * R

NC = 2
NS = 16
NW = NC * NS
K = 80
C = 125
EP = NW * C * K
EW = EP // NW
H = 2
DH = D // H
N_PAD = 10240
ZROWS = N_PAD // NS
NRP = NR
CNT_W = NRP // NS

@functools.cache
def _mesh():
    return plsc.VectorSubcoreMesh(
        core_axis_name="c", subcore_axis_name="s",
        num_cores=NC, num_subcores=NS)


def _idx_body(src_ref, dst_ref, rel_ref, m_ref, seg_ref):
    m_ref[...] = rel_ref[...] * N + src_ref[...]
    seg_ref[...] = dst_ref[...] * R + rel_ref[...]


def _idx_tc(src, dst, rel):
    m, seg = pl.pallas_call(
        _idx_body,
        out_shape=[jax.ShapeDtypeStruct((EP // D, D), jnp.int32)] * 2,
    )(src.reshape(EP // D, D), dst.reshape(EP // D, D),
      rel.reshape(EP // D, D))
    return m, seg


def _inv_body(c_ref, inv_ref):
    tot = c_ref[0] + c_ref[1]
    inv_ref[...] = 1.0 / jnp.maximum(tot, 1.0)


def _inv_tc(cnts):
    inv = pl.pallas_call(
        _inv_body,
        out_shape=jax.ShapeDtypeStruct((NRP // D, D), jnp.float32),
    )(cnts.reshape(NC, NRP // D, D))
    return inv.reshape(NRP)


_BN = 1000


def _transform_body(x_ref, w_ref, ya_ref, yb_ref):
    y = jnp.dot(x_ref[...], w_ref[0], preferred_element_type=jnp.float32)
    ya_ref[0] = y[:, :DH]
    yb_ref[0] = y[:, DH:]


def _transform_tc(h, W):
    ya, yb = pl.pallas_call(
        _transform_body,
        grid=(R, N // _BN),
        in_specs=[
            pl.BlockSpec((_BN, D), lambda r, i: (i, 0)),
            pl.BlockSpec((1, D, D), lambda r, i: (r, 0, 0)),
        ],
        out_specs=[pl.BlockSpec((1, _BN, DH), lambda r, i: (r, i, 0))] * 2,
        out_shape=[jax.ShapeDtypeStruct((R, N, DH), jnp.float32)] * 2,
    )(h, W)
    return ya.reshape(R * N, DH), yb.reshape(R * N, DH)


def _combine_body(pa_ref, pb_ref, x_ref, root_ref, b_ref, o_ref, *, relu):
    msg = jnp.concatenate(
        [pa_ref[0] + pa_ref[1], pb_ref[0] + pb_ref[1]], axis=1)
    v = (msg
         + jnp.dot(x_ref[...], root_ref[...],
                   preferred_element_type=jnp.float32)
         + b_ref[...])
    o_ref[...] = jnp.maximum(v, 0.0) if relu else v


def _combine_tc(pa, pb, h, root, b, relu):
    return pl.pallas_call(
        functools.partial(_combine_body, relu=relu),
        grid=(N // _BN,),
        in_specs=[
            pl.BlockSpec((NC, _BN, DH), lambda i: (0, i, 0)),
            pl.BlockSpec((NC, _BN, DH), lambda i: (0, i, 0)),
            pl.BlockSpec((_BN, D), lambda i: (i, 0)),
            pl.BlockSpec((D, D), lambda i: (0, 0)),
            pl.BlockSpec((1, D), lambda i: (0, 0)),
        ],
        out_specs=pl.BlockSpec((_BN, D), lambda i: (i, 0)),
        out_shape=jax.ShapeDtypeStruct((N, D), jnp.float32),
    )(pa, pb, h, root, b.reshape(1, D))


def _cnt_body(seg_hbm, zc_hbm, out_hbm, seg2d, ones, zbuf, cnt, sem):
    cid = lax.axis_index("c")
    sid = lax.axis_index("s")
    wid = cid * NS + sid

    pltpu.sync_copy(zc_hbm, zbuf)
    pltpu.sync_copy(zbuf, cnt.at[pl.ds(sid * CNT_W, CNT_W)])
    for t in range(K // 16):
        ones[pl.ds(t * 16, 16)] = jnp.full((16,), 1.0, jnp.float32)
    pltpu.sync_copy(seg_hbm.at[wid], seg2d)
    plsc.subcore_barrier()

    def chunk(j, carry):
        pltpu.sync_copy(ones, cnt.at[seg2d.at[j]], add=True)
        return carry

    lax.fori_loop(0, C, chunk, 0)
    plsc.subcore_barrier()
    pltpu.sync_copy(cnt.at[pl.ds(sid * CNT_W, CNT_W)], zbuf)
    pltpu.sync_copy(zbuf,
                    out_hbm.at[pl.ds(cid * NRP + sid * CNT_W, CNT_W)])


@functools.cache
def _cnt_sc():
    return pl.kernel(
        _cnt_body,
        out_type=jax.ShapeDtypeStruct((NC * NRP,), jnp.float32),
        mesh=_mesh(),
        compiler_params=pltpu.CompilerParams(
            needs_layout_passes=False, use_tc_tiling_on_sc=False),
        scratch_types=[
            pltpu.VMEM((C, K), jnp.int32),
            pltpu.VMEM((K,), jnp.float32),
            pltpu.VMEM((CNT_W,), jnp.float32),
            pltpu.VMEM_SHARED((NRP,), jnp.float32),
            pltpu.SemaphoreType.DMA,
        ],
    )


def _edge_body(ya_hbm, yb_hbm, inv_hbm, m_hbm, seg_hbm, dst_hbm, z_hbm,
               outa_hbm, outb_hbm, m2d, seg2d, dst2d, rows, rows1, scal2d,
               acc, sem, sem1, sem2):
    cid = lax.axis_index("c")
    sid = lax.axis_index("s")
    wid = cid * NS + sid

    pltpu.sync_copy(m_hbm.at[wid], m2d)
    pltpu.sync_copy(seg_hbm.at[wid], seg2d)
    pltpu.sync_copy(dst_hbm.at[wid], dst2d)

    for hi, (y_hbm, out_hbm) in enumerate(
            ((ya_hbm, outa_hbm), (yb_hbm, outb_hbm))):
        pltpu.sync_copy(z_hbm, rows)
        for z in range(ZROWS // K):
            pltpu.sync_copy(rows, acc.at[pl.ds(sid * ZROWS + z * K, K)])
        plsc.subcore_barrier()

        def gat(j, buf, bsem):
            return pltpu.async_copy(y_hbm.at[m2d.at[j]], buf, bsem)

        def gat_scal(j):
            return pltpu.async_copy(inv_hbm.at[seg2d.at[j]], scal2d.at[j],
                                    sem2)

        def wait_gat(j, buf, bsem):
            pltpu.make_async_copy(y_hbm.at[m2d.at[j]], buf, bsem).wait()
            if hi == 0:
                pltpu.make_async_copy(
                    inv_hbm.at[seg2d.at[j]], scal2d.at[j], sem2).wait()

        def scale_scatter(j, buf):
            jv = jnp.zeros((16,), jnp.int32) + j

            def blk(g):
                spl = plsc.load_gather(
                    scal2d, [jv, jnp.zeros((16,), jnp.int32) + g])
                for cc in range(DH // 16):
                    buf[g, pl.ds(cc * 16, 16)] = (
                        buf[g, pl.ds(cc * 16, 16)] * spl)

            plsc.parallel_loop(0, K, 1, unroll=32)(blk)
            pltpu.sync_copy(buf, acc.at[dst2d.at[j]], add=True)

        gat(0, rows, sem)
        if hi == 0:
            gat_scal(0)

        def pair(t, carry):
            a = 2 * t
            b = a + 1
            nxt = a + 2
            gat(b, rows1, sem1)
            if hi == 0:
                gat_scal(b)
            wait_gat(a, rows, sem)
            scale_scatter(a, rows)
            gat(nxt, rows, sem)
            if hi == 0:
                gat_scal(nxt)
            wait_gat(b, rows1, sem1)
            scale_scatter(b, rows1)
            return carry

        lax.fori_loop(0, (C - 1) // 2, pair, 0)
        last = C - 1
        wait_gat(last, rows, sem)
        scale_scatter(last, rows)
        plsc.subcore_barrier()
        for z in range(ZROWS // K):
            o = sid * ZROWS + z * K
            pltpu.sync_copy(acc.at[pl.ds(o, K)], rows)
            pltpu.sync_copy(rows, out_hbm.at[pl.ds(cid * N_PAD + o, K)])
        plsc.subcore_barrier()


@functools.cache
def _edge_sc():
    return pl.kernel(
        _edge_body,
        out_type=[jax.ShapeDtypeStruct((NC * N_PAD, DH), jnp.float32)] * 2,
        mesh=_mesh(),
        compiler_params=pltpu.CompilerParams(
            needs_layout_passes=False, use_tc_tiling_on_sc=False),
        scratch_types=[
            pltpu.VMEM((C, K), jnp.int32),
            pltpu.VMEM((C, K), jnp.int32),
            pltpu.VMEM((C, K), jnp.int32),
            pltpu.VMEM((K, DH), jnp.float32),
            pltpu.VMEM((K, DH), jnp.float32),
            pltpu.VMEM((C, K), jnp.float32),
            pltpu.VMEM_SHARED((N_PAD, DH), jnp.float32),
            pltpu.SemaphoreType.DMA,
            pltpu.SemaphoreType.DMA,
            pltpu.SemaphoreType.DMA,
        ],
    )


def kernel(x, edge_index, edge_type, W1, root1, b1, W2, root2, b2):
    src = edge_index[0].astype(jnp.int32)
    dst = edge_index[1].astype(jnp.int32)
    rel = edge_type.astype(jnp.int32)

    m_idx, seg = _idx_tc(src, dst, rel)
    m3 = m_idx.reshape(NW, C, K)
    seg3 = seg.reshape(NW, C, K)
    dst3 = dst.reshape(NW, C, K)

    zc = jnp.zeros((CNT_W,), jnp.float32)
    cnts = _cnt_sc()(seg3, zc)
    inv = _inv_tc(cnts.reshape(NC, NRP))

    zrow = jnp.zeros((K, DH), jnp.float32)

    def layer(h, W, root, b, relu):
        ya, yb = _transform_tc(h, W)
        pa, pb = _edge_sc()(ya, yb, inv, m3, seg3, dst3, zrow)
        pa = pa.reshape(NC, N_PAD, DH)[:, :N]
        pb = pb.reshape(NC, N_PAD, DH)[:, :N]
        return _combine_tc(pa, pb, h, root, b, relu)

    h = layer(x, W1, root1, b1, True)
    return layer(h, W2, root2, b2, False)


# speedup vs baseline: 1.0837x; 1.0837x over previous
import functools

import jax
import jax.numpy as jnp
from jax import lax
from jax.experimental import pallas as pl
from jax.experimental.pallas import tpu as pltpu
from jax.experimental.pallas import tpu_sc as plsc

N = 10000
E = 320000
D = 128
R = 8
NR = N * R

NC = 2
NS = 16
NW = NC * NS
K = 80
C = 125
EP = NW * C * K
EW = EP // NW
H = 2
DH = D // H
N_PAD = 10240
ZROWS = N_PAD // NS
NRP = NR
CNT_W = NRP // NS

@functools.cache
def _mesh():
    return plsc.VectorSubcoreMesh(
        core_axis_name="c", subcore_axis_name="s",
        num_cores=NC, num_subcores=NS)


def _idx_body(src_ref, dst_ref, rel_ref, m_ref, seg_ref):
    m_ref[...] = rel_ref[...] * N + src_ref[...]
    seg_ref[...] = dst_ref[...] * R + rel_ref[...]


def _idx_tc(src, dst, rel):
    m, seg = pl.pallas_call(
        _idx_body,
        out_shape=[jax.ShapeDtypeStruct((EP // D, D), jnp.int32)] * 2,
    )(src.reshape(EP // D, D), dst.reshape(EP // D, D),
      rel.reshape(EP // D, D))
    return m, seg


def _inv_body(c_ref, inv_ref):
    tot = c_ref[0] + c_ref[1]
    inv_ref[...] = 1.0 / jnp.maximum(tot, 1.0)


def _inv_tc(cnts):
    inv = pl.pallas_call(
        _inv_body,
        out_shape=jax.ShapeDtypeStruct((NRP // D, D), jnp.float32),
    )(cnts.reshape(NC, NRP // D, D))
    return inv.reshape(NRP)


_BN = 1000


def _transform_body(x_ref, w_ref, ya_ref, yb_ref):
    y = jnp.dot(x_ref[...], w_ref[0], preferred_element_type=jnp.float32)
    ya_ref[0] = y[:, :DH]
    yb_ref[0] = y[:, DH:]


def _transform_tc(h, W):
    ya, yb = pl.pallas_call(
        _transform_body,
        grid=(R, N // _BN),
        in_specs=[
            pl.BlockSpec((_BN, D), lambda r, i: (i, 0)),
            pl.BlockSpec((1, D, D), lambda r, i: (r, 0, 0)),
        ],
        out_specs=[pl.BlockSpec((1, _BN, DH), lambda r, i: (r, i, 0))] * 2,
        out_shape=[jax.ShapeDtypeStruct((R, N, DH), jnp.float32)] * 2,
    )(h, W)
    return ya.reshape(R * N, DH), yb.reshape(R * N, DH)


def _combine_body(pa_ref, pb_ref, x_ref, root_ref, b_ref, o_ref, *, relu):
    msg = jnp.concatenate(
        [pa_ref[0] + pa_ref[1], pb_ref[0] + pb_ref[1]], axis=1)
    v = (msg
         + jnp.dot(x_ref[...], root_ref[...],
                   preferred_element_type=jnp.float32)
         + b_ref[...])
    o_ref[...] = jnp.maximum(v, 0.0) if relu else v


def _combine_tc(pa, pb, h, root, b, relu):
    return pl.pallas_call(
        functools.partial(_combine_body, relu=relu),
        grid=(N // _BN,),
        in_specs=[
            pl.BlockSpec((NC, _BN, DH), lambda i: (0, i, 0)),
            pl.BlockSpec((NC, _BN, DH), lambda i: (0, i, 0)),
            pl.BlockSpec((_BN, D), lambda i: (i, 0)),
            pl.BlockSpec((D, D), lambda i: (0, 0)),
            pl.BlockSpec((1, D), lambda i: (0, 0)),
        ],
        out_specs=pl.BlockSpec((_BN, D), lambda i: (i, 0)),
        out_shape=jax.ShapeDtypeStruct((N, D), jnp.float32),
    )(pa, pb, h, root, b.reshape(1, D))


def _cnt_body(seg_hbm, zc_hbm, out_hbm, seg2d, ones, zbuf, cnt, sem):
    cid = lax.axis_index("c")
    sid = lax.axis_index("s")
    wid = cid * NS + sid

    pltpu.sync_copy(zc_hbm, zbuf)
    pltpu.sync_copy(zbuf, cnt.at[pl.ds(sid * CNT_W, CNT_W)])
    for t in range(K // 16):
        ones[pl.ds(t * 16, 16)] = jnp.full((16,), 1.0, jnp.float32)
    pltpu.sync_copy(seg_hbm.at[wid], seg2d)
    plsc.subcore_barrier()

    def chunk(j, carry):
        pltpu.sync_copy(ones, cnt.at[seg2d.at[j]], add=True)
        return carry

    lax.fori_loop(0, C, chunk, 0)
    plsc.subcore_barrier()
    pltpu.sync_copy(cnt.at[pl.ds(sid * CNT_W, CNT_W)], zbuf)
    pltpu.sync_copy(zbuf,
                    out_hbm.at[pl.ds(cid * NRP + sid * CNT_W, CNT_W)])


@functools.cache
def _cnt_sc():
    return pl.kernel(
        _cnt_body,
        out_type=jax.ShapeDtypeStruct((NC * NRP,), jnp.float32),
        mesh=_mesh(),
        compiler_params=pltpu.CompilerParams(
            needs_layout_passes=False, use_tc_tiling_on_sc=False),
        scratch_types=[
            pltpu.VMEM((C, K), jnp.int32),
            pltpu.VMEM((K,), jnp.float32),
            pltpu.VMEM((CNT_W,), jnp.float32),
            pltpu.VMEM_SHARED((NRP,), jnp.float32),
            pltpu.SemaphoreType.DMA,
        ],
    )


def _edge_body(ya_hbm, yb_hbm, inv_hbm, m_hbm, seg_hbm, dst_hbm, z_hbm,
               outa_hbm, outb_hbm, m2d, seg2d, dst2d, rows, rows1, robuf,
               scal2d, acc, sem, sem1, sem2):
    cid = lax.axis_index("c")
    sid = lax.axis_index("s")
    wid = cid * NS + sid

    pltpu.sync_copy(m_hbm.at[wid], m2d)
    pltpu.sync_copy(seg_hbm.at[wid], seg2d)
    pltpu.sync_copy(dst_hbm.at[wid], dst2d)

    for hi, (y_hbm, out_hbm) in enumerate(
            ((ya_hbm, outa_hbm), (yb_hbm, outb_hbm))):
        pltpu.sync_copy(z_hbm, rows)
        for z in range(ZROWS // K):
            pltpu.sync_copy(rows, acc.at[pl.ds(sid * ZROWS + z * K, K)])
        plsc.subcore_barrier()

        def gat(j, buf, bsem):
            return pltpu.async_copy(y_hbm.at[m2d.at[j]], buf, bsem)

        def gat_scal(j):
            return pltpu.async_copy(inv_hbm.at[seg2d.at[j]], scal2d.at[j],
                                    sem2)

        def wait_gat(j, buf, bsem):
            pltpu.make_async_copy(y_hbm.at[m2d.at[j]], buf, bsem).wait()
            if hi == 0:
                pltpu.make_async_copy(
                    inv_hbm.at[seg2d.at[j]], scal2d.at[j], sem2).wait()

        def scale_scatter(j, buf):
            jv = jnp.zeros((16,), jnp.int32) + j

            def blk(g):
                spl = plsc.load_gather(
                    scal2d, [jv, jnp.zeros((16,), jnp.int32) + g])
                for cc in range(DH // 16):
                    buf[g, pl.ds(cc * 16, 16)] = (
                        buf[g, pl.ds(cc * 16, 16)] * spl)

            plsc.parallel_loop(0, K, 1, unroll=16)(blk)
            pltpu.sync_copy(buf, acc.at[dst2d.at[j]], add=True)

        gat(0, rows, sem)
        if hi == 0:
            gat_scal(0)

        def pair(t, carry):
            a = 2 * t
            b = a + 1
            nxt = a + 2
            gat(b, rows1, sem1)
            if hi == 0:
                gat_scal(b)
            wait_gat(a, rows, sem)
            scale_scatter(a, rows)
            gat(nxt, rows, sem)
            if hi == 0:
                gat_scal(nxt)
            wait_gat(b, rows1, sem1)
            scale_scatter(b, rows1)
            return carry

        lax.fori_loop(0, (C - 1) // 2, pair, 0)
        last = C - 1
        wait_gat(last, rows, sem)
        scale_scatter(last, rows)
        plsc.subcore_barrier()
        RD = N // NS // 5
        for z in range(5):
            o = sid * (N // NS) + z * RD
            pltpu.sync_copy(acc.at[pl.ds(o, RD)], robuf)
            pltpu.sync_copy(robuf, out_hbm.at[pl.ds(cid * N + o, RD)])
        plsc.subcore_barrier()


@functools.cache
def _edge_sc():
    return pl.kernel(
        _edge_body,
        out_type=[jax.ShapeDtypeStruct((NC * N, DH), jnp.float32)] * 2,
        mesh=_mesh(),
        compiler_params=pltpu.CompilerParams(
            needs_layout_passes=False, use_tc_tiling_on_sc=False),
        scratch_types=[
            pltpu.VMEM((C, K), jnp.int32),
            pltpu.VMEM((C, K), jnp.int32),
            pltpu.VMEM((C, K), jnp.int32),
            pltpu.VMEM((K, DH), jnp.float32),
            pltpu.VMEM((K, DH), jnp.float32),
            pltpu.VMEM((125, DH), jnp.float32),
            pltpu.VMEM((C, K), jnp.float32),
            pltpu.VMEM_SHARED((N_PAD, DH), jnp.float32),
            pltpu.SemaphoreType.DMA,
            pltpu.SemaphoreType.DMA,
            pltpu.SemaphoreType.DMA,
        ],
    )


def kernel(x, edge_index, edge_type, W1, root1, b1, W2, root2, b2):
    src = edge_index[0].astype(jnp.int32)
    dst = edge_index[1].astype(jnp.int32)
    rel = edge_type.astype(jnp.int32)

    m_idx, seg = _idx_tc(src, dst, rel)
    m3 = m_idx.reshape(NW, C, K)
    seg3 = seg.reshape(NW, C, K)
    dst3 = dst.reshape(NW, C, K)

    zc = jnp.zeros((CNT_W,), jnp.float32)
    cnts = _cnt_sc()(seg3, zc)
    inv = _inv_tc(cnts.reshape(NC, NRP))

    zrow = jnp.zeros((K, DH), jnp.float32)

    def layer(h, W, root, b, relu):
        ya, yb = _transform_tc(h, W)
        pa, pb = _edge_sc()(ya, yb, inv, m3, seg3, dst3, zrow)
        return _combine_tc(pa.reshape(NC, N, DH), pb.reshape(NC, N, DH),
                           h, root, b, relu)

    h = layer(x, W1, root1, b1, True)
    return layer(h, W2, root2, b2, False)
